# baseline (speedup 1.0000x reference)
import functools

import jax
import jax.numpy as jnp
import numpy as np
from jax import lax
from jax.experimental import pallas as pl
from jax.experimental.pallas import tpu as pltpu
from jax.experimental.pallas import tpu_sc as plsc

B, V, L = 4096, 20, 50
VOCAB, D, NCLS = 100000, 64, 100
NIDX = V * L
NCHUNK = 8
CW = NIDX // NCHUNK
CWP = 128
DW = D // 2
NC, NS = 2, 16
NW = NC * NS
P = B // NW
RU = 5
NP = 128
_HI = -65536

_PERM = np.empty(2 * D, np.int32)
for _c in range(2 * D):
    _t, _r = divmod(_c, D)
    _g, _k = divmod(_r, 32)
    _PERM[_c] = _t * D + _g * 32 + (2 * _k if _k < 16 else 2 * (_k - 16) + 1)


def _pool_body(emb_hbm, didx_hbm, pidx_hbm, out_hbm, ix_v,
               rb0, rb1, rb2, rb3, rb4, rb5, rb6, rb7,
               rb8, rb9, rb10, rb11, rb12, rb13, rb14, rb15, obuf_v,
               sg0, sg1, sg2, sg3, sg4, sg5, sg6, sg7,
               sg8, sg9, sg10, sg11, sg12, sg13, sg14, sg15, semi):
    rbufs = (rb0, rb1, rb2, rb3, rb4, rb5, rb6, rb7,
             rb8, rb9, rb10, rb11, rb12, rb13, rb14, rb15)
    sems = (sg0, sg1, sg2, sg3, sg4, sg5, sg6, sg7,
            sg8, sg9, sg10, sg11, sg12, sg13, sg14, sg15)
    wid = lax.axis_index("s") * NC + lax.axis_index("c")
    base = wid * P

    def idx_start(b, p):
        pltpu.async_copy(didx_hbm.at[pl.ds(b, 8)], ix_v.at[p, 0], semi)
        pltpu.async_copy(pidx_hbm.at[pl.ds(b, 8)], ix_v.at[p, 1], semi)

    def idx_wait(p):
        pltpu.make_async_copy(
            didx_hbm.at[pl.ds(base, 8)], ix_v.at[p, 0], semi).wait()
        pltpu.make_async_copy(
            pidx_hbm.at[pl.ds(base, 8)], ix_v.at[p, 1], semi).wait()

    def reduce_to_obuf(s, t, rbuf):
        acc = (jnp.zeros((16,), jnp.float32),) * 4

        def red(r, a):
            a = list(a)
            for u in range(RU):
                rr = r * RU + u
                for g in range(2):
                    w = plsc.bitcast(rbuf[rr, pl.ds(g * 32, 32)], jnp.int32)
                    a[2 * g] = a[2 * g] + plsc.bitcast(w << 16, jnp.float32)
                    a[2 * g + 1] = a[2 * g + 1] + plsc.bitcast(
                        w & _HI, jnp.float32)
            return tuple(a)

        acc = lax.fori_loop(0, CW // RU, red, acc)
        for q in range(4):
            obuf_v[s, pl.ds(t * D + q * 16, 16)] = acc[q] * (1.0 / L)

    def process_quad(p, s0):
        jobs = [(sm, t) for sm in range(8) for t in range(2)]
        cps = [None] * 16
        for J, (sm, t) in enumerate(jobs):
            cps[J] = pltpu.async_copy(
                emb_hbm.at[ix_v.at[p, t, sm, 0]], rbufs[J], sems[J])
        for c in range(1, NCHUNK):
            for J, (sm, t) in enumerate(jobs):
                cps[J].wait()
                cps[J] = pltpu.async_copy(
                    emb_hbm.at[ix_v.at[p, t, sm, c]], rbufs[J], sems[J],
                    add=True)
        for J, (sm, t) in enumerate(jobs):
            cps[J].wait()
            reduce_to_obuf(s0 + sm, t, rbufs[J])

    def oct_body(i, carry):
        s0 = 16 * i
        idx_wait(0)
        idx_start(base + s0 + 8, 1)
        process_quad(0, s0)
        idx_wait(1)
        idx_start(base + lax.min(s0 + 16, P - 8), 0)
        process_quad(1, s0 + 8)
        return carry

    idx_start(base, 0)
    lax.fori_loop(0, P // 16, oct_body, 0)
    idx_wait(0)
    pltpu.sync_copy(obuf_v, out_hbm.at[pl.ds(base, P)])


def _matmul_body(x_ref, w_ref, b_ref, o_ref):
    o_ref[...] = lax.dot_general(
        x_ref[...], w_ref[...], (((1,), (1,)), ((), ())),
        preferred_element_type=jnp.float32,
    ) + b_ref[...]


@jax.jit
def kernel(diagnoses_idx, procedures_idx, emb, W, b):
    pad3 = ((0, 0), (0, 0), (0, CWP - CW))
    didx = jnp.pad(diagnoses_idx.reshape(B, NCHUNK, CW).astype(jnp.int32),
                   pad3)
    pidx = jnp.pad(procedures_idx.reshape(B, NCHUNK, CW).astype(jnp.int32),
                   pad3)
    emb_bf = emb.astype(jnp.bfloat16)

    pooled = pl.kernel(
        _pool_body,
        out_type=jax.ShapeDtypeStruct((B, 2 * D), jnp.float32),
        mesh=plsc.VectorSubcoreMesh(
            core_axis_name="c", subcore_axis_name="s",
            num_cores=NC, num_subcores=NS),
        scratch_types=(
            [pltpu.VMEM((2, 2, 8, NCHUNK, CWP), jnp.int32)]
            + [pltpu.VMEM((CWP, D), jnp.bfloat16) for _ in range(16)]
            + [pltpu.VMEM((P, 2 * D), jnp.float32)]
            + [pltpu.SemaphoreType.DMA for _ in range(17)]
        ),
        compiler_params=pltpu.CompilerParams(
            use_tc_tiling_on_sc=False, needs_layout_passes=False),
    )(emb_bf, didx, pidx)

    w_perm = W[:, _PERM]
    w_pad = jnp.zeros((NP, 2 * D), jnp.float32).at[:NCLS].set(w_perm)
    b_pad = jnp.zeros((1, NP), jnp.float32).at[0, :NCLS].set(b)

    rows_per_blk = 256
    out = pl.pallas_call(
        _matmul_body,
        grid=(B // rows_per_blk,),
        in_specs=[
            pl.BlockSpec((rows_per_blk, 2 * D), lambda i: (i, 0)),
            pl.BlockSpec((NP, 2 * D), lambda i: (0, 0)),
            pl.BlockSpec((1, NP), lambda i: (0, 0)),
        ],
        out_specs=pl.BlockSpec((rows_per_blk, NP), lambda i: (i, 0)),
        out_shape=jax.ShapeDtypeStruct((B, NP), jnp.float32),
    )(pooled, w_pad, b_pad)
    return out[:, :NCLS]


# speedup vs baseline: 4.0031x; 4.0031x over previous
import functools

import jax
import jax.numpy as jnp
import numpy as np
from jax import lax
from jax.experimental import pallas as pl
from jax.experimental.pallas import tpu as pltpu
from jax.experimental.pallas import tpu_sc as plsc

B, V, L = 4096, 20, 50
VOCAB, D, NCLS = 100000, 64, 100
NIDX = V * L
NCHUNK = 8
CW = NIDX // NCHUNK
DW = D // 2
NC, NS = 2, 16
NW = NC * NS
P = B // NW
RU = 5
NP = 128
_HI = -65536

_PERM = np.empty(2 * D, np.int32)
for _c in range(2 * D):
    _t, _r = divmod(_c, D)
    _g, _k = divmod(_r, 32)
    _PERM[_c] = _t * D + _g * 32 + (2 * _k if _k < 16 else 2 * (_k - 16) + 1)


def _pool_body(emb_hbm, didx_hbm, pidx_hbm, out_hbm, ix_v,
               rb0, rb1, rb2, rb3, rb4, rb5, rb6, rb7,
               rb8, rb9, rb10, rb11, rb12, rb13, rb14, rb15, obuf_v,
               sg0, sg1, sg2, sg3, sg4, sg5, sg6, sg7,
               sg8, sg9, sg10, sg11, sg12, sg13, sg14, sg15, semi):
    rbufs = (rb0, rb1, rb2, rb3, rb4, rb5, rb6, rb7,
             rb8, rb9, rb10, rb11, rb12, rb13, rb14, rb15)
    sems = (sg0, sg1, sg2, sg3, sg4, sg5, sg6, sg7,
            sg8, sg9, sg10, sg11, sg12, sg13, sg14, sg15)
    wid = lax.axis_index("s") * NC + lax.axis_index("c")
    base = wid * P

    def idx_start(b, p):
        pltpu.async_copy(didx_hbm.at[pl.ds(b, 4)], ix_v.at[p, 0], semi)
        pltpu.async_copy(pidx_hbm.at[pl.ds(b, 4)], ix_v.at[p, 1], semi)

    def idx_wait(p):
        pltpu.make_async_copy(
            didx_hbm.at[pl.ds(base, 4)], ix_v.at[p, 0], semi).wait()
        pltpu.make_async_copy(
            pidx_hbm.at[pl.ds(base, 4)], ix_v.at[p, 1], semi).wait()

    def reduce_to_obuf(s, t, rbuf):
        acc = (jnp.zeros((16,), jnp.float32),) * 4

        def red(r, a):
            a = list(a)
            for u in range(RU):
                rr = r * RU + u
                for g in range(2):
                    w = plsc.bitcast(rbuf[rr, pl.ds(g * 32, 32)], jnp.int32)
                    a[2 * g] = a[2 * g] + plsc.bitcast(w << 16, jnp.float32)
                    a[2 * g + 1] = a[2 * g + 1] + plsc.bitcast(
                        w & _HI, jnp.float32)
            return tuple(a)

        acc = lax.fori_loop(0, CW // RU, red, acc)
        for q in range(4):
            obuf_v[s, pl.ds(t * D + q * 16, 16)] = acc[q] * (1.0 / L)

    gjobs = [(sm, t) for sm in range(4) for t in range(2)]

    def issue_c0(p, off):
        for J, (sm, t) in enumerate(gjobs):
            pltpu.async_copy(
                emb_hbm.at[ix_v.at[p, t, sm, 0]], rbufs[off + J],
                sems[off + J])

    def step_chains(p, off):
        cps = [pltpu.make_async_copy(
            emb_hbm.at[ix_v.at[p, t, sm, 0]], rbufs[off + J], sems[off + J])
            for J, (sm, t) in enumerate(gjobs)]
        for c in range(1, NCHUNK):
            for J, (sm, t) in enumerate(gjobs):
                cps[J].wait()
                cps[J] = pltpu.async_copy(
                    emb_hbm.at[ix_v.at[p, t, sm, c]], rbufs[off + J],
                    sems[off + J], add=True)
        for J in range(8):
            cps[J].wait()

    def reduce_group(s0, off):
        for J, (sm, t) in enumerate(gjobs):
            reduce_to_obuf(s0 + sm, t, rbufs[off + J])

    def body(i, carry):
        s0 = 8 * i
        step_chains(0, 0)
        idx_wait(1)
        issue_c0(1, 8)
        idx_start(base + lax.min(s0 + 8, P - 4), 0)
        reduce_group(s0, 0)
        step_chains(1, 8)
        idx_wait(0)
        issue_c0(0, 0)
        idx_start(base + lax.min(s0 + 12, P - 4), 1)
        reduce_group(s0 + 4, 8)
        return carry

    idx_start(base, 0)
    idx_wait(0)
    issue_c0(0, 0)
    idx_start(base + 4, 1)
    lax.fori_loop(0, P // 8, body, 0)
    for J, (sm, t) in enumerate(gjobs):
        pltpu.make_async_copy(
            emb_hbm.at[ix_v.at[0, t, sm, 0]], rbufs[J], sems[J]).wait()
    idx_wait(1)
    pltpu.sync_copy(obuf_v, out_hbm.at[pl.ds(base, P)])


def _matmul_body(x_ref, w_ref, b_ref, o_ref):
    o_ref[...] = lax.dot_general(
        x_ref[...], w_ref[...], (((1,), (1,)), ((), ())),
        preferred_element_type=jnp.float32,
    ) + b_ref[...]


@jax.jit
def kernel(diagnoses_idx, procedures_idx, emb, W, b):
    didx = diagnoses_idx.reshape(B, NCHUNK, CW).astype(jnp.int32)
    pidx = procedures_idx.reshape(B, NCHUNK, CW).astype(jnp.int32)
    emb_bf = emb.astype(jnp.bfloat16)

    pooled = pl.kernel(
        _pool_body,
        out_type=jax.ShapeDtypeStruct((B, 2 * D), jnp.float32),
        mesh=plsc.VectorSubcoreMesh(
            core_axis_name="c", subcore_axis_name="s",
            num_cores=NC, num_subcores=NS),
        scratch_types=(
            [pltpu.VMEM((2, 2, 4, NCHUNK, CW), jnp.int32)]
            + [pltpu.VMEM((CW, D), jnp.bfloat16) for _ in range(16)]
            + [pltpu.VMEM((P, 2 * D), jnp.float32)]
            + [pltpu.SemaphoreType.DMA for _ in range(17)]
        ),
        compiler_params=pltpu.CompilerParams(
            use_tc_tiling_on_sc=False, needs_layout_passes=False),
    )(emb_bf, didx, pidx)

    w_perm = W[:, _PERM]
    w_pad = jnp.zeros((NP, 2 * D), jnp.float32).at[:NCLS].set(w_perm)
    b_pad = jnp.zeros((1, NP), jnp.float32).at[0, :NCLS].set(b)

    rows_per_blk = 256
    out = pl.pallas_call(
        _matmul_body,
        grid=(B // rows_per_blk,),
        in_specs=[
            pl.BlockSpec((rows_per_blk, 2 * D), lambda i: (i, 0)),
            pl.BlockSpec((NP, 2 * D), lambda i: (0, 0)),
            pl.BlockSpec((1, NP), lambda i: (0, 0)),
        ],
        out_specs=pl.BlockSpec((rows_per_blk, NP), lambda i: (i, 0)),
        out_shape=jax.ShapeDtypeStruct((B, NP), jnp.float32),
    )(pooled, w_pad, b_pad)
    return out[:, :NCLS]


# speedup vs baseline: 4.0079x; 1.0012x over previous
import jax
import jax.numpy as jnp
import numpy as np
from jax import lax
from jax.experimental import pallas as pl
from jax.experimental.pallas import tpu as pltpu
from jax.experimental.pallas import tpu_sc as plsc

B, V, L = 4096, 20, 50
VOCAB, D, NCLS = 100000, 64, 100
NIDX = V * L
NCHUNK = 8
CW = NIDX // NCHUNK
DW = D // 2
NC, NS = 2, 16
NW = NC * NS
P = B // NW
RU = 5
NP = 128
_HI = -65536

_PERM = np.empty(2 * D, np.int32)
for _c in range(2 * D):
    _t, _r = divmod(_c, D)
    _g, _k = divmod(_r, 32)
    _PERM[_c] = _t * D + _g * 32 + (2 * _k if _k < 16 else 2 * (_k - 16) + 1)


def _pool_body(emb_hbm, didx_hbm, pidx_hbm, out_hbm, ix_v,
               rb0, rb1, rb2, rb3, rb4, rb5, rb6, rb7,
               rb8, rb9, rb10, rb11, rb12, rb13, rb14, rb15, obuf_v,
               sg0, sg1, sg2, sg3, sg4, sg5, sg6, sg7,
               sg8, sg9, sg10, sg11, sg12, sg13, sg14, sg15, semi):
    rbufs = (rb0, rb1, rb2, rb3, rb4, rb5, rb6, rb7,
             rb8, rb9, rb10, rb11, rb12, rb13, rb14, rb15)
    sems = (sg0, sg1, sg2, sg3, sg4, sg5, sg6, sg7,
            sg8, sg9, sg10, sg11, sg12, sg13, sg14, sg15)
    wid = lax.axis_index("s") * NC + lax.axis_index("c")
    base = wid * P

    def idx_start(b, p):
        pltpu.async_copy(didx_hbm.at[pl.ds(b, 4)], ix_v.at[p, 0], semi)
        pltpu.async_copy(pidx_hbm.at[pl.ds(b, 4)], ix_v.at[p, 1], semi)

    def idx_wait(p):
        pltpu.make_async_copy(
            didx_hbm.at[pl.ds(base, 4)], ix_v.at[p, 0], semi).wait()
        pltpu.make_async_copy(
            pidx_hbm.at[pl.ds(base, 4)], ix_v.at[p, 1], semi).wait()

    def reduce_to_obuf(s, t, rbuf):
        acc = (jnp.zeros((16,), jnp.float32),) * 4

        def red(r, a):
            a = list(a)
            for u in range(RU):
                rr = r * RU + u
                for g in range(2):
                    w = plsc.bitcast(rbuf[rr, pl.ds(g * 32, 32)], jnp.int32)
                    a[2 * g] = a[2 * g] + plsc.bitcast(w << 16, jnp.float32)
                    a[2 * g + 1] = a[2 * g + 1] + plsc.bitcast(
                        w & _HI, jnp.float32)
            return tuple(a)

        acc = lax.fori_loop(0, CW // RU, red, acc)
        for q in range(4):
            obuf_v[s, pl.ds(t * D + q * 16, 16)] = acc[q] * (1.0 / L)

    gjobs = [(sm, t) for sm in range(4) for t in range(2)]

    def issue_c0(p, off):
        for J, (sm, t) in enumerate(gjobs):
            pltpu.async_copy(
                emb_hbm.at[ix_v.at[p, t, sm, 0]], rbufs[off + J],
                sems[off + J])

    def step_chains(p, off):
        cps = [pltpu.make_async_copy(
            emb_hbm.at[ix_v.at[p, t, sm, 0]], rbufs[off + J], sems[off + J])
            for J, (sm, t) in enumerate(gjobs)]
        for c in range(1, NCHUNK):
            for J, (sm, t) in enumerate(gjobs):
                cps[J].wait()
                cps[J] = pltpu.async_copy(
                    emb_hbm.at[ix_v.at[p, t, sm, c]], rbufs[off + J],
                    sems[off + J], add=True)
        for J in range(8):
            cps[J].wait()

    def reduce_group(s0, off):
        for J, (sm, t) in enumerate(gjobs):
            reduce_to_obuf(s0 + sm, t, rbufs[off + J])

    def body(i, carry):
        s0 = 8 * i
        step_chains(0, 0)
        idx_wait(1)
        issue_c0(1, 8)
        idx_start(base + lax.min(s0 + 8, P - 4), 0)
        reduce_group(s0, 0)
        step_chains(1, 8)
        idx_wait(0)
        issue_c0(0, 0)
        idx_start(base + lax.min(s0 + 12, P - 4), 1)
        reduce_group(s0 + 4, 8)
        return carry

    idx_start(base, 0)
    idx_wait(0)
    issue_c0(0, 0)
    idx_start(base + 4, 1)
    lax.fori_loop(0, P // 8, body, 0)
    for J, (sm, t) in enumerate(gjobs):
        pltpu.make_async_copy(
            emb_hbm.at[ix_v.at[0, t, sm, 0]], rbufs[J], sems[J]).wait()
    idx_wait(1)
    pltpu.sync_copy(obuf_v, out_hbm.at[pl.ds(base, P)])


def _matmul_body(x_ref, w_ref, b_ref, o_ref):
    o_ref[...] = lax.dot_general(
        x_ref[...], w_ref[...], (((1,), (1,)), ((), ())),
        preferred_element_type=jnp.float32,
    ) + b_ref[...]


@jax.jit
def kernel(diagnoses_idx, procedures_idx, emb, W, b):
    didx = diagnoses_idx.reshape(B, NCHUNK, CW).astype(jnp.int32)
    pidx = procedures_idx.reshape(B, NCHUNK, CW).astype(jnp.int32)
    emb_bf = emb.astype(jnp.bfloat16)

    pooled = pl.kernel(
        _pool_body,
        out_type=jax.ShapeDtypeStruct((B, 2 * D), jnp.float32),
        mesh=plsc.VectorSubcoreMesh(
            core_axis_name="c", subcore_axis_name="s",
            num_cores=NC, num_subcores=NS),
        scratch_types=(
            [pltpu.VMEM((2, 2, 4, NCHUNK, CW), jnp.int32)]
            + [pltpu.VMEM((CW, D), jnp.bfloat16) for _ in range(16)]
            + [pltpu.VMEM((P, 2 * D), jnp.float32)]
            + [pltpu.SemaphoreType.DMA for _ in range(17)]
        ),
        compiler_params=pltpu.CompilerParams(
            use_tc_tiling_on_sc=False, needs_layout_passes=False),
    )(emb_bf, didx, pidx)

    w_perm = W[:, _PERM]
    w_pad = jnp.zeros((NP, 2 * D), jnp.float32).at[:NCLS].set(w_perm)
    b_pad = jnp.zeros((1, NP), jnp.float32).at[0, :NCLS].set(b)

    rows_per_blk = 256
    out = pl.pallas_call(
        _matmul_body,
        grid=(B // rows_per_blk,),
        in_specs=[
            pl.BlockSpec((rows_per_blk, 2 * D), lambda i: (i, 0)),
            pl.BlockSpec((NP, 2 * D), lambda i: (0, 0)),
            pl.BlockSpec((1, NP), lambda i: (0, 0)),
        ],
        out_specs=pl.BlockSpec((rows_per_blk, NP), lambda i: (i, 0)),
        out_shape=jax.ShapeDtypeStruct((B, NP), jnp.float32),
    )(pooled, w_pad, b_pad)
    return out[:, :NCLS]
